# Initial kernel scaffold; baseline (speedup 1.0000x reference)
import jax
import jax.numpy as jnp
from jax.experimental import pallas as pl


def kernel(edge_vec, edge_index, pos, les_strain, batch, cell_volume):
    raise NotImplementedError("write your pallas kernel here")


# speedup vs baseline: 1.7866x; 1.7866x over previous
import functools

import jax
import jax.numpy as jnp
from jax import lax
from jax.experimental import pallas as pl
from jax.experimental.pallas import tpu as pltpu
from jax.experimental.pallas import tpu_sc as plsc

N_EDGES = 1600000
N_NODES = 50000
NBATCH = 16

NC = 2
NS = 16
NW = NC * NS
EPW = N_EDGES // NW
CHUNK = 400
NCHUNKS = EPW // CHUNK
SUB = 100
KSUB = CHUNK // SUB
GROUPS = CHUNK // 16
IDX_ROWS = N_EDGES // SUB

NPAD = 50176
RPT = NPAD // NS
NBLK = 8
BLK = NPAD // NBLK


def _sc_body(ev_hbm, src_hbm, dst_hbm, z8_hbm,
             accf_out, accv_out,
             evb, srcb, dstb, ua, ub, uc, accf, accv):
    cid = lax.axis_index("c")
    sid = lax.axis_index("s")
    wid = cid * NS + sid

    pltpu.sync_copy(z8_hbm, accf.at[pl.ds(sid * RPT, RPT)])
    pltpu.sync_copy(z8_hbm, accv.at[pl.ds(sid * RPT, RPT)])

    lanes = lax.iota(jnp.int32, 16)
    zf = jnp.zeros((16,), jnp.float32)
    c0 = jnp.zeros((16,), jnp.int32)
    c1 = jnp.full((16,), 1, jnp.int32)
    c2 = jnp.full((16,), 2, jnp.int32)
    c3 = jnp.full((16,), 3, jnp.int32)
    c4 = jnp.full((16,), 4, jnp.int32)
    c5 = jnp.full((16,), 5, jnp.int32)
    c6 = jnp.full((16,), 6, jnp.int32)
    c7 = jnp.full((16,), 7, jnp.int32)

    for g in range(GROUPS):
        rows = lanes + (g * 16)
        for cp in (c3, c4, c5, c6, c7):
            plsc.store_scatter(ua, [rows, cp], zf)
            plsc.store_scatter(ub, [rows, cp], zf)
        plsc.store_scatter(uc, [rows, c6], zf)
        plsc.store_scatter(uc, [rows, c7], zf)

    plsc.subcore_barrier()

    def chunk_body(i, carry):
        ebase = wid * EPW + i * CHUNK
        rbase = wid * (EPW // SUB) + i * KSUB
        pltpu.sync_copy(ev_hbm.at[pl.ds(ebase, CHUNK)], evb)
        pltpu.sync_copy(src_hbm.at[pl.ds(rbase, KSUB)], srcb)
        pltpu.sync_copy(dst_hbm.at[pl.ds(rbase, KSUB)], dstb)
        for g in range(GROUPS):
            rows = lanes + (g * 16)
            ex = plsc.load_gather(evb, [rows, c0])
            ey = plsc.load_gather(evb, [rows, c1])
            ez = plsc.load_gather(evb, [rows, c2])
            r2 = ex * ex + ey * ey + ez * ez
            s = -jnp.exp(r2 * -0.5)
            fx = s * ex
            fy = s * ey
            fz = s * ez
            plsc.store_scatter(ua, [rows, c0], fx)
            plsc.store_scatter(ua, [rows, c1], fy)
            plsc.store_scatter(ua, [rows, c2], fz)
            plsc.store_scatter(ub, [rows, c0], -fx)
            plsc.store_scatter(ub, [rows, c1], -fy)
            plsc.store_scatter(ub, [rows, c2], -fz)
            plsc.store_scatter(uc, [rows, c0], fx * ex)
            plsc.store_scatter(uc, [rows, c1], fy * ey)
            plsc.store_scatter(uc, [rows, c2], fz * ez)
            plsc.store_scatter(uc, [rows, c3], ex * fy)
            plsc.store_scatter(uc, [rows, c4], ey * fz)
            plsc.store_scatter(uc, [rows, c5], ez * fx)
        for j in range(KSUB):
            pltpu.sync_copy(ua.at[pl.ds(j * SUB, SUB)], accf.at[srcb.at[j]], add=True)
            pltpu.sync_copy(ub.at[pl.ds(j * SUB, SUB)], accf.at[dstb.at[j]], add=True)
            pltpu.sync_copy(uc.at[pl.ds(j * SUB, SUB)], accv.at[dstb.at[j]], add=True)
        return carry

    lax.fori_loop(0, NCHUNKS, chunk_body, 0)

    plsc.subcore_barrier()

    pltpu.sync_copy(accf.at[pl.ds(sid * RPT, RPT)],
                    accf_out.at[cid, pl.ds(sid * RPT, RPT)])
    pltpu.sync_copy(accv.at[pl.ds(sid * RPT, RPT)],
                    accv_out.at[cid, pl.ds(sid * RPT, RPT)])


_sc_scatter = functools.partial(
    pl.kernel,
    out_type=(
        jax.ShapeDtypeStruct((NC, NPAD, 8), jnp.float32),
        jax.ShapeDtypeStruct((NC, NPAD, 8), jnp.float32),
    ),
    mesh=plsc.VectorSubcoreMesh(core_axis_name="c", subcore_axis_name="s",
                                num_cores=NC),
    compiler_params=pltpu.CompilerParams(needs_layout_passes=False,
                                         use_tc_tiling_on_sc=False),
    scratch_types=[
        pltpu.VMEM((CHUNK, 3), jnp.float32),
        pltpu.VMEM((KSUB, SUB), jnp.int32),
        pltpu.VMEM((KSUB, SUB), jnp.int32),
        pltpu.VMEM((CHUNK, 8), jnp.float32),
        pltpu.VMEM((CHUNK, 8), jnp.float32),
        pltpu.VMEM((CHUNK, 8), jnp.float32),
        pltpu.VMEM_SHARED((NPAD, 8), jnp.float32),
        pltpu.VMEM_SHARED((NPAD, 8), jnp.float32),
    ],
)(_sc_body)


def _tc_body(accf_ref, accv_ref, pos_ref, batch_ref, strain_ref, vol_ref,
             force_ref, stress_ref):
    accf = accf_ref[0] + accf_ref[1]
    force_ref[...] = accf[:, :3] + 0.01 * jnp.sin(pos_ref[...])
    accv = accv_ref[0] + accv_ref[1]
    b = batch_ref[0, 0]
    oh = (b[None, :] == lax.broadcasted_iota(jnp.int32, (NBATCH, BLK), 0))
    part = jnp.dot(oh.astype(jnp.float32), accv[:, :6],
                   preferred_element_type=jnp.float32)
    vol = vol_ref[...]

    @pl.when(pl.program_id(0) == 0)
    def _():
        s9 = strain_ref[...]
        lr = jnp.concatenate(
            [s9[:, 0:1], s9[:, 4:5], s9[:, 8:9],
             s9[:, 1:2], s9[:, 5:6], s9[:, 2:3]], axis=1)
        stress_ref[...] = lr * (-2.0) / vol

    stress_ref[...] += -part / vol


_tc_epilogue = pl.pallas_call(
    _tc_body,
    grid=(NBLK,),
    in_specs=[
        pl.BlockSpec((NC, BLK, 8), lambda i: (0, i, 0)),
        pl.BlockSpec((NC, BLK, 8), lambda i: (0, i, 0)),
        pl.BlockSpec((BLK, 3), lambda i: (i, 0)),
        pl.BlockSpec((1, 1, BLK), lambda i: (i, 0, 0)),
        pl.BlockSpec((NBATCH, 9), lambda i: (0, 0)),
        pl.BlockSpec((NBATCH, 1), lambda i: (0, 0)),
    ],
    out_specs=[
        pl.BlockSpec((BLK, 3), lambda i: (i, 0)),
        pl.BlockSpec((NBATCH, 6), lambda i: (0, 0)),
    ],
    out_shape=[
        jax.ShapeDtypeStruct((NPAD, 3), jnp.float32),
        jax.ShapeDtypeStruct((NBATCH, 6), jnp.float32),
    ],
)


def kernel(edge_vec, edge_index, pos, les_strain, batch, cell_volume):
    src2d = edge_index[0].astype(jnp.int32).reshape(IDX_ROWS, SUB)
    dst2d = edge_index[1].astype(jnp.int32).reshape(IDX_ROWS, SUB)
    z8 = jnp.zeros((RPT, 8), jnp.float32)
    accf, accv = _sc_scatter(edge_vec, src2d, dst2d, z8)

    pos_pad = jnp.pad(pos, ((0, NPAD - N_NODES), (0, 0)))
    batch_pad = jnp.pad(batch.astype(jnp.int32),
                        (0, NPAD - N_NODES)).reshape(NBLK, 1, BLK)
    strain9 = les_strain.reshape(NBATCH, 9)
    vol2 = cell_volume.reshape(NBATCH, 1)
    force_pad, stress = _tc_epilogue(accf, accv, pos_pad, batch_pad,
                                     strain9, vol2)
    return force_pad[:N_NODES], stress


# speedup vs baseline: 1.9304x; 1.0805x over previous
import functools

import jax
import jax.numpy as jnp
from jax import lax
from jax.experimental import pallas as pl
from jax.experimental.pallas import tpu as pltpu
from jax.experimental.pallas import tpu_sc as plsc

N_EDGES = 1600000
N_NODES = 50000
NBATCH = 16

NC = 2
NS = 16
NW = NC * NS
EPW = N_EDGES // NW
CHUNK = 400
NCHUNKS = EPW // CHUNK
SUB = 100
KSUB = CHUNK // SUB
GROUPS = CHUNK // 16
IDX_ROWS = N_EDGES // SUB

NPAD = 50176
RPT = NPAD // NS
NBLK = 8
BLK = NPAD // NBLK


def _sc_body(ev_hbm, src_hbm, dst_hbm, z8_hbm,
             accf_out, accv_out,
             evb, srcb, dstb, ua, ub, uc, accf, accv):
    cid = lax.axis_index("c")
    sid = lax.axis_index("s")
    wid = cid * NS + sid

    pltpu.sync_copy(z8_hbm, accf.at[pl.ds(sid * RPT, RPT)])
    pltpu.sync_copy(z8_hbm, accv.at[pl.ds(sid * RPT, RPT)])

    lanes = lax.iota(jnp.int32, 16)
    lanes3 = lanes * 3
    zf = jnp.zeros((16,), jnp.float32)
    c0 = jnp.zeros((16,), jnp.int32)
    c1 = jnp.full((16,), 1, jnp.int32)
    c2 = jnp.full((16,), 2, jnp.int32)
    c3 = jnp.full((16,), 3, jnp.int32)
    c4 = jnp.full((16,), 4, jnp.int32)
    c5 = jnp.full((16,), 5, jnp.int32)
    c6 = jnp.full((16,), 6, jnp.int32)
    c7 = jnp.full((16,), 7, jnp.int32)

    for g in range(GROUPS):
        rows = lanes + (g * 16)
        for cp in (c3, c4, c5, c6, c7):
            plsc.store_scatter(ua, [rows, cp], zf)
            plsc.store_scatter(ub, [rows, cp], zf)
        plsc.store_scatter(uc, [rows, c6], zf)
        plsc.store_scatter(uc, [rows, c7], zf)

    plsc.subcore_barrier()

    def chunk_body(i, carry):
        ebase = wid * EPW + i * CHUNK
        rbase = wid * (EPW // SUB) + i * KSUB
        pltpu.sync_copy(ev_hbm.at[pl.ds(ebase * 3, CHUNK * 3)], evb)
        pltpu.sync_copy(src_hbm.at[pl.ds(rbase, KSUB)], srcb)
        pltpu.sync_copy(dst_hbm.at[pl.ds(rbase, KSUB)], dstb)
        for g in range(GROUPS):
            rows = lanes + (g * 16)
            ex = plsc.load_gather(evb, [lanes3 + (g * 48)])
            ey = plsc.load_gather(evb, [lanes3 + (g * 48 + 1)])
            ez = plsc.load_gather(evb, [lanes3 + (g * 48 + 2)])
            r2 = ex * ex + ey * ey + ez * ez
            s = -jnp.exp(r2 * -0.5)
            fx = s * ex
            fy = s * ey
            fz = s * ez
            plsc.store_scatter(ua, [rows, c0], fx)
            plsc.store_scatter(ua, [rows, c1], fy)
            plsc.store_scatter(ua, [rows, c2], fz)
            plsc.store_scatter(ub, [rows, c0], -fx)
            plsc.store_scatter(ub, [rows, c1], -fy)
            plsc.store_scatter(ub, [rows, c2], -fz)
            plsc.store_scatter(uc, [rows, c0], fx * ex)
            plsc.store_scatter(uc, [rows, c1], fy * ey)
            plsc.store_scatter(uc, [rows, c2], fz * ez)
            plsc.store_scatter(uc, [rows, c3], ex * fy)
            plsc.store_scatter(uc, [rows, c4], ey * fz)
            plsc.store_scatter(uc, [rows, c5], ez * fx)
        for j in range(KSUB):
            pltpu.sync_copy(ua.at[pl.ds(j * SUB, SUB)], accf.at[srcb.at[j]], add=True)
            pltpu.sync_copy(ub.at[pl.ds(j * SUB, SUB)], accf.at[dstb.at[j]], add=True)
            pltpu.sync_copy(uc.at[pl.ds(j * SUB, SUB)], accv.at[dstb.at[j]], add=True)
        return carry

    lax.fori_loop(0, NCHUNKS, chunk_body, 0)

    plsc.subcore_barrier()

    pltpu.sync_copy(accf.at[pl.ds(sid * RPT, RPT)],
                    accf_out.at[cid, pl.ds(sid * RPT, RPT)])
    pltpu.sync_copy(accv.at[pl.ds(sid * RPT, RPT)],
                    accv_out.at[cid, pl.ds(sid * RPT, RPT)])


_sc_scatter = functools.partial(
    pl.kernel,
    out_type=(
        jax.ShapeDtypeStruct((NC, NPAD, 8), jnp.float32),
        jax.ShapeDtypeStruct((NC, NPAD, 8), jnp.float32),
    ),
    mesh=plsc.VectorSubcoreMesh(core_axis_name="c", subcore_axis_name="s",
                                num_cores=NC),
    compiler_params=pltpu.CompilerParams(needs_layout_passes=False,
                                         use_tc_tiling_on_sc=False),
    scratch_types=[
        pltpu.VMEM((CHUNK * 3,), jnp.float32),
        pltpu.VMEM((KSUB, SUB), jnp.int32),
        pltpu.VMEM((KSUB, SUB), jnp.int32),
        pltpu.VMEM((CHUNK, 8), jnp.float32),
        pltpu.VMEM((CHUNK, 8), jnp.float32),
        pltpu.VMEM((CHUNK, 8), jnp.float32),
        pltpu.VMEM_SHARED((NPAD, 8), jnp.float32),
        pltpu.VMEM_SHARED((NPAD, 8), jnp.float32),
    ],
)(_sc_body)


def _tc_body(accf_ref, accv_ref, pos_ref, batch_ref, strain_ref, vol_ref,
             force_ref, stress_ref):
    accf = accf_ref[0] + accf_ref[1]
    force_ref[...] = accf[:, :3] + 0.01 * jnp.sin(pos_ref[...])
    accv = accv_ref[0] + accv_ref[1]
    b = batch_ref[0, 0]
    oh = (b[None, :] == lax.broadcasted_iota(jnp.int32, (NBATCH, BLK), 0))
    part = jnp.dot(oh.astype(jnp.float32), accv[:, :6],
                   preferred_element_type=jnp.float32)
    vol = vol_ref[...]

    @pl.when(pl.program_id(0) == 0)
    def _():
        s9 = strain_ref[...]
        lr = jnp.concatenate(
            [s9[:, 0:1], s9[:, 4:5], s9[:, 8:9],
             s9[:, 1:2], s9[:, 5:6], s9[:, 2:3]], axis=1)
        stress_ref[...] = lr * (-2.0) / vol

    stress_ref[...] += -part / vol


_tc_epilogue = pl.pallas_call(
    _tc_body,
    grid=(NBLK,),
    in_specs=[
        pl.BlockSpec((NC, BLK, 8), lambda i: (0, i, 0)),
        pl.BlockSpec((NC, BLK, 8), lambda i: (0, i, 0)),
        pl.BlockSpec((BLK, 3), lambda i: (i, 0)),
        pl.BlockSpec((1, 1, BLK), lambda i: (i, 0, 0)),
        pl.BlockSpec((NBATCH, 9), lambda i: (0, 0)),
        pl.BlockSpec((NBATCH, 1), lambda i: (0, 0)),
    ],
    out_specs=[
        pl.BlockSpec((BLK, 3), lambda i: (i, 0)),
        pl.BlockSpec((NBATCH, 6), lambda i: (0, 0)),
    ],
    out_shape=[
        jax.ShapeDtypeStruct((NPAD, 3), jnp.float32),
        jax.ShapeDtypeStruct((NBATCH, 6), jnp.float32),
    ],
)


def kernel(edge_vec, edge_index, pos, les_strain, batch, cell_volume):
    ev1 = edge_vec.reshape(N_EDGES * 3)
    src2d = edge_index[0].astype(jnp.int32).reshape(IDX_ROWS, SUB)
    dst2d = edge_index[1].astype(jnp.int32).reshape(IDX_ROWS, SUB)
    z8 = jnp.zeros((RPT, 8), jnp.float32)
    accf, accv = _sc_scatter(ev1, src2d, dst2d, z8)

    pos_pad = jnp.pad(pos, ((0, NPAD - N_NODES), (0, 0)))
    batch_pad = jnp.pad(batch.astype(jnp.int32),
                        (0, NPAD - N_NODES)).reshape(NBLK, 1, BLK)
    strain9 = les_strain.reshape(NBATCH, 9)
    vol2 = cell_volume.reshape(NBATCH, 1)
    force_pad, stress = _tc_epilogue(accf, accv, pos_pad, batch_pad,
                                     strain9, vol2)
    return force_pad[:N_NODES], stress


# speedup vs baseline: 8.7811x; 4.5489x over previous
import functools

import jax
import jax.numpy as jnp
from jax import lax
from jax.experimental import pallas as pl
from jax.experimental.pallas import tpu as pltpu
from jax.experimental.pallas import tpu_sc as plsc

N_EDGES = 1600000
N_NODES = 50000
NBATCH = 16

NC = 2
NS = 16
NW = NC * NS
EPW = N_EDGES // NW
CHUNK = 400
NCHUNKS = EPW // CHUNK
SUB = 100
KSUB = CHUNK // SUB
GROUPS = CHUNK // 16
IDX_ROWS = N_EDGES // SUB

NPAD = 50176
RPT = NPAD // NS
NBLK = 8
BLK = NPAD // NBLK


def _sc_body(ev_hbm, src_hbm, dst_hbm, z8_hbm,
             accf_out, accv_out,
             evxb, evyb, evzb, srcb, dstb, ua, ub, uc, accf, accv):
    cid = lax.axis_index("c")
    sid = lax.axis_index("s")
    wid = cid * NS + sid

    pltpu.sync_copy(z8_hbm, accf.at[pl.ds(sid * RPT, RPT)])
    pltpu.sync_copy(z8_hbm, accv.at[pl.ds(sid * RPT, RPT)])

    lanes = lax.iota(jnp.int32, 16)
    zf = jnp.zeros((16,), jnp.float32)
    c0 = jnp.zeros((16,), jnp.int32)
    c1 = jnp.full((16,), 1, jnp.int32)
    c2 = jnp.full((16,), 2, jnp.int32)
    c3 = jnp.full((16,), 3, jnp.int32)
    c4 = jnp.full((16,), 4, jnp.int32)
    c5 = jnp.full((16,), 5, jnp.int32)
    c6 = jnp.full((16,), 6, jnp.int32)
    c7 = jnp.full((16,), 7, jnp.int32)

    for g in range(GROUPS):
        rows = lanes + (g * 16)
        for cp in (c3, c4, c5, c6, c7):
            plsc.store_scatter(ua, [rows, cp], zf)
            plsc.store_scatter(ub, [rows, cp], zf)
        plsc.store_scatter(uc, [rows, c6], zf)
        plsc.store_scatter(uc, [rows, c7], zf)

    plsc.subcore_barrier()

    def chunk_body(i, carry):
        ebase = wid * EPW + i * CHUNK
        rbase = wid * (EPW // SUB) + i * KSUB
        pltpu.sync_copy(ev_hbm.at[0, pl.ds(ebase, CHUNK)], evxb)
        pltpu.sync_copy(ev_hbm.at[1, pl.ds(ebase, CHUNK)], evyb)
        pltpu.sync_copy(ev_hbm.at[2, pl.ds(ebase, CHUNK)], evzb)
        pltpu.sync_copy(src_hbm.at[pl.ds(rbase, KSUB)], srcb)
        pltpu.sync_copy(dst_hbm.at[pl.ds(rbase, KSUB)], dstb)
        for g in range(GROUPS):
            rows = lanes + (g * 16)
            ex = evxb[pl.ds(g * 16, 16)]
            ey = evyb[pl.ds(g * 16, 16)]
            ez = evzb[pl.ds(g * 16, 16)]
            r2 = ex * ex + ey * ey + ez * ez
            s = -jnp.exp(r2 * -0.5)
            fx = s * ex
            fy = s * ey
            fz = s * ez
            plsc.store_scatter(ua, [rows, c0], fx)
            plsc.store_scatter(ua, [rows, c1], fy)
            plsc.store_scatter(ua, [rows, c2], fz)
            plsc.store_scatter(ub, [rows, c0], -fx)
            plsc.store_scatter(ub, [rows, c1], -fy)
            plsc.store_scatter(ub, [rows, c2], -fz)
            plsc.store_scatter(uc, [rows, c0], fx * ex)
            plsc.store_scatter(uc, [rows, c1], fy * ey)
            plsc.store_scatter(uc, [rows, c2], fz * ez)
            plsc.store_scatter(uc, [rows, c3], ex * fy)
            plsc.store_scatter(uc, [rows, c4], ey * fz)
            plsc.store_scatter(uc, [rows, c5], ez * fx)
        for j in range(KSUB):
            pltpu.sync_copy(ua.at[pl.ds(j * SUB, SUB)], accf.at[srcb.at[j]], add=True)
            pltpu.sync_copy(ub.at[pl.ds(j * SUB, SUB)], accf.at[dstb.at[j]], add=True)
            pltpu.sync_copy(uc.at[pl.ds(j * SUB, SUB)], accv.at[dstb.at[j]], add=True)
        return carry

    lax.fori_loop(0, NCHUNKS, chunk_body, 0)

    plsc.subcore_barrier()

    pltpu.sync_copy(accf.at[pl.ds(sid * RPT, RPT)],
                    accf_out.at[cid, pl.ds(sid * RPT, RPT)])
    pltpu.sync_copy(accv.at[pl.ds(sid * RPT, RPT)],
                    accv_out.at[cid, pl.ds(sid * RPT, RPT)])


_sc_scatter = functools.partial(
    pl.kernel,
    out_type=(
        jax.ShapeDtypeStruct((NC, NPAD, 8), jnp.float32),
        jax.ShapeDtypeStruct((NC, NPAD, 8), jnp.float32),
    ),
    mesh=plsc.VectorSubcoreMesh(core_axis_name="c", subcore_axis_name="s",
                                num_cores=NC),
    compiler_params=pltpu.CompilerParams(needs_layout_passes=False,
                                         use_tc_tiling_on_sc=False),
    scratch_types=[
        pltpu.VMEM((CHUNK,), jnp.float32),
        pltpu.VMEM((CHUNK,), jnp.float32),
        pltpu.VMEM((CHUNK,), jnp.float32),
        pltpu.VMEM((KSUB, SUB), jnp.int32),
        pltpu.VMEM((KSUB, SUB), jnp.int32),
        pltpu.VMEM((CHUNK, 8), jnp.float32),
        pltpu.VMEM((CHUNK, 8), jnp.float32),
        pltpu.VMEM((CHUNK, 8), jnp.float32),
        pltpu.VMEM_SHARED((NPAD, 8), jnp.float32),
        pltpu.VMEM_SHARED((NPAD, 8), jnp.float32),
    ],
)(_sc_body)


def _tc_body(accf_ref, accv_ref, pos_ref, batch_ref, strain_ref, vol_ref,
             force_ref, stress_ref):
    accf = accf_ref[0] + accf_ref[1]
    force_ref[...] = accf[:, :3] + 0.01 * jnp.sin(pos_ref[...])
    accv = accv_ref[0] + accv_ref[1]
    b = batch_ref[0, 0]
    oh = (b[None, :] == lax.broadcasted_iota(jnp.int32, (NBATCH, BLK), 0))
    part = jnp.dot(oh.astype(jnp.float32), accv[:, :6],
                   preferred_element_type=jnp.float32)
    vol = vol_ref[...]

    @pl.when(pl.program_id(0) == 0)
    def _():
        s9 = strain_ref[...]
        lr = jnp.concatenate(
            [s9[:, 0:1], s9[:, 4:5], s9[:, 8:9],
             s9[:, 1:2], s9[:, 5:6], s9[:, 2:3]], axis=1)
        stress_ref[...] = lr * (-2.0) / vol

    stress_ref[...] += -part / vol


_tc_epilogue = pl.pallas_call(
    _tc_body,
    grid=(NBLK,),
    in_specs=[
        pl.BlockSpec((NC, BLK, 8), lambda i: (0, i, 0)),
        pl.BlockSpec((NC, BLK, 8), lambda i: (0, i, 0)),
        pl.BlockSpec((BLK, 3), lambda i: (i, 0)),
        pl.BlockSpec((1, 1, BLK), lambda i: (i, 0, 0)),
        pl.BlockSpec((NBATCH, 9), lambda i: (0, 0)),
        pl.BlockSpec((NBATCH, 1), lambda i: (0, 0)),
    ],
    out_specs=[
        pl.BlockSpec((BLK, 3), lambda i: (i, 0)),
        pl.BlockSpec((NBATCH, 6), lambda i: (0, 0)),
    ],
    out_shape=[
        jax.ShapeDtypeStruct((NPAD, 3), jnp.float32),
        jax.ShapeDtypeStruct((NBATCH, 6), jnp.float32),
    ],
)


def kernel(edge_vec, edge_index, pos, les_strain, batch, cell_volume):
    evt = edge_vec.T
    src2d = edge_index[0].astype(jnp.int32).reshape(IDX_ROWS, SUB)
    dst2d = edge_index[1].astype(jnp.int32).reshape(IDX_ROWS, SUB)
    z8 = jnp.zeros((RPT, 8), jnp.float32)
    accf, accv = _sc_scatter(evt, src2d, dst2d, z8)

    pos_pad = jnp.pad(pos, ((0, NPAD - N_NODES), (0, 0)))
    batch_pad = jnp.pad(batch.astype(jnp.int32),
                        (0, NPAD - N_NODES)).reshape(NBLK, 1, BLK)
    strain9 = les_strain.reshape(NBATCH, 9)
    vol2 = cell_volume.reshape(NBATCH, 1)
    force_pad, stress = _tc_epilogue(accf, accv, pos_pad, batch_pad,
                                     strain9, vol2)
    return force_pad[:N_NODES], stress


# speedup vs baseline: 10.9011x; 1.2414x over previous
import functools

import jax
import jax.numpy as jnp
from jax import lax
from jax.experimental import pallas as pl
from jax.experimental.pallas import tpu as pltpu
from jax.experimental.pallas import tpu_sc as plsc

N_EDGES = 1600000
N_NODES = 50000
NBATCH = 16

NC = 2
NS = 16
NW = NC * NS
EPW = N_EDGES // NW
CHUNK = 400
NCHUNKS = EPW // CHUNK
SUB = 100
KSUB = CHUNK // SUB
GROUPS = CHUNK // 16
IDX_ROWS = N_EDGES // SUB
RPW = EPW // SUB

NPAD = 50176
RPT = NPAD // NS
NBLK = 8
BLK = NPAD // NBLK


def _sc_body(ev_hbm, src_hbm, dst_hbm, z8_hbm, z16_hbm,
             accf_out, accd_out,
             evx0, evx1, evx2, evy0, evy1, evy2, evz0, evz1, evz2,
             srcb0, srcb1, srcb2, dstb0, dstb1, dstb2,
             ua0, ua1, ua2, ud0, ud1, ud2,
             sin0, sin1, sin2,
             accf, accd):
    cid = lax.axis_index("c")
    sid = lax.axis_index("s")
    wid = cid * NS + sid

    evx = (evx0, evx1, evx2)
    evy = (evy0, evy1, evy2)
    evz = (evz0, evz1, evz2)
    srcb = (srcb0, srcb1, srcb2)
    dstb = (dstb0, dstb1, dstb2)
    ua = (ua0, ua1, ua2)
    ud = (ud0, ud1, ud2)
    sin = (sin0, sin1, sin2)

    pltpu.sync_copy(z8_hbm, accf.at[pl.ds(sid * RPT, RPT)])
    pltpu.sync_copy(z16_hbm, accd.at[pl.ds(sid * RPT, RPT)])

    lanes = lax.iota(jnp.int32, 16)
    zf = jnp.zeros((16,), jnp.float32)
    cc = [jnp.full((16,), c, jnp.int32) for c in range(16)]

    def zero_body(g, carry):
        rows = lanes + g * 16
        for k in range(3):
            for c in range(3, 8):
                plsc.store_scatter(ua[k], [rows, cc[c]], zf)
            for c in range(9, 16):
                plsc.store_scatter(ud[k], [rows, cc[c]], zf)
        return carry

    lax.fori_loop(0, GROUPS, zero_body, 0)

    def issue_inputs(m, k):
        ebase = wid * EPW + m * CHUNK
        rbase = wid * RPW + m * KSUB
        pltpu.async_copy(ev_hbm.at[0, pl.ds(ebase, CHUNK)], evx[k], sin[k])
        pltpu.async_copy(ev_hbm.at[1, pl.ds(ebase, CHUNK)], evy[k], sin[k])
        pltpu.async_copy(ev_hbm.at[2, pl.ds(ebase, CHUNK)], evz[k], sin[k])
        pltpu.async_copy(src_hbm.at[pl.ds(rbase, KSUB)], srcb[k], sin[k])
        pltpu.async_copy(dst_hbm.at[pl.ds(rbase, KSUB)], dstb[k], sin[k])

    def wait_inputs(k):
        pltpu.make_async_copy(ev_hbm.at[0, pl.ds(0, CHUNK)], evx[k], sin[k]).wait()
        pltpu.make_async_copy(ev_hbm.at[1, pl.ds(0, CHUNK)], evy[k], sin[k]).wait()
        pltpu.make_async_copy(ev_hbm.at[2, pl.ds(0, CHUNK)], evz[k], sin[k]).wait()
        pltpu.make_async_copy(src_hbm.at[pl.ds(0, KSUB)], srcb[k], sin[k]).wait()
        pltpu.make_async_copy(dst_hbm.at[pl.ds(0, KSUB)], dstb[k], sin[k]).wait()

    def compute(k):
        for g in range(GROUPS):
            rows = lanes + g * 16
            ex = evx[k][pl.ds(g * 16, 16)]
            ey = evy[k][pl.ds(g * 16, 16)]
            ez = evz[k][pl.ds(g * 16, 16)]
            r2 = ex * ex + ey * ey + ez * ez
            s = -jnp.exp(r2 * -0.5)
            fx = s * ex
            fy = s * ey
            fz = s * ez
            plsc.store_scatter(ua[k], [rows, cc[0]], fx)
            plsc.store_scatter(ua[k], [rows, cc[1]], fy)
            plsc.store_scatter(ua[k], [rows, cc[2]], fz)
            plsc.store_scatter(ud[k], [rows, cc[0]], -fx)
            plsc.store_scatter(ud[k], [rows, cc[1]], -fy)
            plsc.store_scatter(ud[k], [rows, cc[2]], -fz)
            plsc.store_scatter(ud[k], [rows, cc[3]], fx * ex)
            plsc.store_scatter(ud[k], [rows, cc[4]], fy * ey)
            plsc.store_scatter(ud[k], [rows, cc[5]], fz * ez)
            plsc.store_scatter(ud[k], [rows, cc[6]], ex * fy)
            plsc.store_scatter(ud[k], [rows, cc[7]], ey * fz)
            plsc.store_scatter(ud[k], [rows, cc[8]], ez * fx)

    def scatters(k):
        for j in range(KSUB):
            pltpu.sync_copy(ua[k].at[pl.ds(j * SUB, SUB)],
                            accf.at[srcb[k].at[j]], add=True)
            pltpu.sync_copy(ud[k].at[pl.ds(j * SUB, SUB)],
                            accd.at[dstb[k].at[j]], add=True)

    def half(m, k, prefetch):
        wait_inputs(k)
        if prefetch:
            issue_inputs(m + 2, (k + 2) % 3)
        compute(k)
        scatters(k)

    issue_inputs(0, 0)
    issue_inputs(1, 1)

    plsc.subcore_barrier()

    def body(t, carry):
        m = 3 * t
        half(m, 0, True)
        half(m + 1, 1, True)
        half(m + 2, 2, True)
        return carry

    lax.fori_loop(0, (NCHUNKS - 2) // 3, body, 0)

    half(NCHUNKS - 2, 0, False)
    half(NCHUNKS - 1, 1, False)

    plsc.subcore_barrier()

    pltpu.sync_copy(accf.at[pl.ds(sid * RPT, RPT)],
                    accf_out.at[cid, pl.ds(sid * RPT, RPT)])
    pltpu.sync_copy(accd.at[pl.ds(sid * RPT, RPT)],
                    accd_out.at[cid, pl.ds(sid * RPT, RPT)])


_sc_scatter = functools.partial(
    pl.kernel,
    out_type=(
        jax.ShapeDtypeStruct((NC, NPAD, 8), jnp.float32),
        jax.ShapeDtypeStruct((NC, NPAD, 16), jnp.float32),
    ),
    mesh=plsc.VectorSubcoreMesh(core_axis_name="c", subcore_axis_name="s",
                                num_cores=NC),
    compiler_params=pltpu.CompilerParams(needs_layout_passes=False,
                                         use_tc_tiling_on_sc=False),
    scratch_types=(
        [pltpu.VMEM((CHUNK,), jnp.float32) for _ in range(9)]
        + [pltpu.VMEM((KSUB, SUB), jnp.int32) for _ in range(6)]
        + [pltpu.VMEM((CHUNK, 8), jnp.float32) for _ in range(3)]
        + [pltpu.VMEM((CHUNK, 16), jnp.float32) for _ in range(3)]
        + [pltpu.SemaphoreType.DMA for _ in range(3)]
        + [pltpu.VMEM_SHARED((NPAD, 8), jnp.float32),
           pltpu.VMEM_SHARED((NPAD, 16), jnp.float32)]
    ),
)(_sc_body)


def _tc_body(accf_ref, accd_ref, pos_ref, batch_ref, strain_ref, vol_ref,
             force_ref, stress_ref):
    accf = accf_ref[0] + accf_ref[1]
    accd = accd_ref[0] + accd_ref[1]
    force_ref[...] = accf[:, :3] + accd[:, :3] + 0.01 * jnp.sin(pos_ref[...])
    b = batch_ref[0, 0]
    oh = (b[None, :] == lax.broadcasted_iota(jnp.int32, (NBATCH, BLK), 0))
    part = jnp.dot(oh.astype(jnp.float32), accd[:, 3:9],
                   preferred_element_type=jnp.float32)
    vol = vol_ref[...]

    @pl.when(pl.program_id(0) == 0)
    def _():
        s9 = strain_ref[...]
        lr = jnp.concatenate(
            [s9[:, 0:1], s9[:, 4:5], s9[:, 8:9],
             s9[:, 1:2], s9[:, 5:6], s9[:, 2:3]], axis=1)
        stress_ref[...] = lr * (-2.0) / vol

    stress_ref[...] += -part / vol


_tc_epilogue = pl.pallas_call(
    _tc_body,
    grid=(NBLK,),
    in_specs=[
        pl.BlockSpec((NC, BLK, 8), lambda i: (0, i, 0)),
        pl.BlockSpec((NC, BLK, 16), lambda i: (0, i, 0)),
        pl.BlockSpec((BLK, 3), lambda i: (i, 0)),
        pl.BlockSpec((1, 1, BLK), lambda i: (i, 0, 0)),
        pl.BlockSpec((NBATCH, 9), lambda i: (0, 0)),
        pl.BlockSpec((NBATCH, 1), lambda i: (0, 0)),
    ],
    out_specs=[
        pl.BlockSpec((BLK, 3), lambda i: (i, 0)),
        pl.BlockSpec((NBATCH, 6), lambda i: (0, 0)),
    ],
    out_shape=[
        jax.ShapeDtypeStruct((NPAD, 3), jnp.float32),
        jax.ShapeDtypeStruct((NBATCH, 6), jnp.float32),
    ],
)


def kernel(edge_vec, edge_index, pos, les_strain, batch, cell_volume):
    evt = edge_vec.T
    src2d = edge_index[0].astype(jnp.int32).reshape(IDX_ROWS, SUB)
    dst2d = edge_index[1].astype(jnp.int32).reshape(IDX_ROWS, SUB)
    z8 = jnp.zeros((RPT, 8), jnp.float32)
    z16 = jnp.zeros((RPT, 16), jnp.float32)
    accf, accd = _sc_scatter(evt, src2d, dst2d, z8, z16)

    pos_pad = jnp.pad(pos, ((0, NPAD - N_NODES), (0, 0)))
    batch_pad = jnp.pad(batch.astype(jnp.int32),
                        (0, NPAD - N_NODES)).reshape(NBLK, 1, BLK)
    strain9 = les_strain.reshape(NBATCH, 9)
    vol2 = cell_volume.reshape(NBATCH, 1)
    force_pad, stress = _tc_epilogue(accf, accd, pos_pad, batch_pad,
                                     strain9, vol2)
    return force_pad[:N_NODES], stress


# speedup vs baseline: 12.0632x; 1.1066x over previous
import functools

import jax
import jax.numpy as jnp
from jax import lax
from jax.experimental import pallas as pl
from jax.experimental.pallas import tpu as pltpu
from jax.experimental.pallas import tpu_sc as plsc

N_EDGES = 1600000
N_NODES = 50000
NBATCH = 16

NC = 2
NS = 16
NW = NC * NS
EPW = N_EDGES // NW
CHUNK = 400
NCHUNKS = EPW // CHUNK
SUB = 80
KSUB = CHUNK // SUB
GROUPS = CHUNK // 16

NPAD = 50176
RPT = NPAD // NS
NBLK = 8
BLK = NPAD // NBLK


def _sc_body(ev_hbm, ei_hbm, z8_hbm, z16_hbm,
             accf_out, accd_out,
             evx0, evx1, evx2, evy0, evy1, evy2, evz0, evz1, evz2,
             srcb0, srcb1, srcb2, dstb0, dstb1, dstb2,
             ua0, ua1, ua2, ud0, ud1, ud2,
             sin0, sin1, sin2,
             accf, accd):
    cid = lax.axis_index("c")
    sid = lax.axis_index("s")
    wid = cid * NS + sid

    evx = (evx0, evx1, evx2)
    evy = (evy0, evy1, evy2)
    evz = (evz0, evz1, evz2)
    srcb = (srcb0, srcb1, srcb2)
    dstb = (dstb0, dstb1, dstb2)
    ua = (ua0, ua1, ua2)
    ud = (ud0, ud1, ud2)
    sin = (sin0, sin1, sin2)

    pltpu.sync_copy(z8_hbm, accf.at[pl.ds(sid * RPT, RPT)])
    pltpu.sync_copy(z16_hbm, accd.at[pl.ds(sid * RPT, RPT)])

    lanes = lax.iota(jnp.int32, 16)
    zf = jnp.zeros((16,), jnp.float32)
    cc = [jnp.full((16,), c, jnp.int32) for c in range(16)]

    def zero_body(g, carry):
        rows = lanes + g * 16
        for k in range(3):
            for c in range(3, 8):
                plsc.store_scatter(ua[k], [rows, cc[c]], zf)
            for c in range(9, 16):
                plsc.store_scatter(ud[k], [rows, cc[c]], zf)
        return carry

    lax.fori_loop(0, GROUPS, zero_body, 0)

    def issue_inputs(m, k):
        ebase = wid * EPW + m * CHUNK
        pltpu.async_copy(ev_hbm.at[0, pl.ds(ebase, CHUNK)], evx[k], sin[k])
        pltpu.async_copy(ev_hbm.at[1, pl.ds(ebase, CHUNK)], evy[k], sin[k])
        pltpu.async_copy(ev_hbm.at[2, pl.ds(ebase, CHUNK)], evz[k], sin[k])
        for j in range(KSUB):
            pltpu.async_copy(ei_hbm.at[0, pl.ds(ebase + j * SUB, SUB)],
                             srcb[k].at[j], sin[k])
            pltpu.async_copy(ei_hbm.at[1, pl.ds(ebase + j * SUB, SUB)],
                             dstb[k].at[j], sin[k])

    def wait_inputs(k):
        pltpu.make_async_copy(ev_hbm.at[0, pl.ds(0, CHUNK)], evx[k], sin[k]).wait()
        pltpu.make_async_copy(ev_hbm.at[1, pl.ds(0, CHUNK)], evy[k], sin[k]).wait()
        pltpu.make_async_copy(ev_hbm.at[2, pl.ds(0, CHUNK)], evz[k], sin[k]).wait()
        for j in range(KSUB):
            pltpu.make_async_copy(ei_hbm.at[0, pl.ds(0, SUB)],
                                  srcb[k].at[j], sin[k]).wait()
            pltpu.make_async_copy(ei_hbm.at[1, pl.ds(0, SUB)],
                                  dstb[k].at[j], sin[k]).wait()

    def compute(k):
        for g in range(GROUPS):
            rows = lanes + g * 16
            ex = evx[k][pl.ds(g * 16, 16)]
            ey = evy[k][pl.ds(g * 16, 16)]
            ez = evz[k][pl.ds(g * 16, 16)]
            r2 = ex * ex + ey * ey + ez * ez
            s = -jnp.exp(r2 * -0.5)
            fx = s * ex
            fy = s * ey
            fz = s * ez
            plsc.store_scatter(ua[k], [rows, cc[0]], fx)
            plsc.store_scatter(ua[k], [rows, cc[1]], fy)
            plsc.store_scatter(ua[k], [rows, cc[2]], fz)
            plsc.store_scatter(ud[k], [rows, cc[0]], -fx)
            plsc.store_scatter(ud[k], [rows, cc[1]], -fy)
            plsc.store_scatter(ud[k], [rows, cc[2]], -fz)
            plsc.store_scatter(ud[k], [rows, cc[3]], fx * ex)
            plsc.store_scatter(ud[k], [rows, cc[4]], fy * ey)
            plsc.store_scatter(ud[k], [rows, cc[5]], fz * ez)
            plsc.store_scatter(ud[k], [rows, cc[6]], ex * fy)
            plsc.store_scatter(ud[k], [rows, cc[7]], ey * fz)
            plsc.store_scatter(ud[k], [rows, cc[8]], ez * fx)

    def scatters(k):
        for j in range(KSUB):
            pltpu.sync_copy(ua[k].at[pl.ds(j * SUB, SUB)],
                            accf.at[srcb[k].at[j]], add=True)
            pltpu.sync_copy(ud[k].at[pl.ds(j * SUB, SUB)],
                            accd.at[dstb[k].at[j]], add=True)

    def half(m, k, prefetch):
        wait_inputs(k)
        if prefetch:
            issue_inputs(m + 2, (k + 2) % 3)
        compute(k)
        scatters(k)

    issue_inputs(0, 0)
    issue_inputs(1, 1)

    plsc.subcore_barrier()

    def body(t, carry):
        m = 3 * t
        half(m, 0, True)
        half(m + 1, 1, True)
        half(m + 2, 2, True)
        return carry

    lax.fori_loop(0, (NCHUNKS - 2) // 3, body, 0)

    half(NCHUNKS - 2, 0, False)
    half(NCHUNKS - 1, 1, False)

    plsc.subcore_barrier()

    pltpu.sync_copy(accf.at[pl.ds(sid * RPT, RPT)],
                    accf_out.at[cid, pl.ds(sid * RPT, RPT)])
    pltpu.sync_copy(accd.at[pl.ds(sid * RPT, RPT)],
                    accd_out.at[cid, pl.ds(sid * RPT, RPT)])


_sc_scatter = functools.partial(
    pl.kernel,
    out_type=(
        jax.ShapeDtypeStruct((NC, NPAD, 8), jnp.float32),
        jax.ShapeDtypeStruct((NC, NPAD, 16), jnp.float32),
    ),
    mesh=plsc.VectorSubcoreMesh(core_axis_name="c", subcore_axis_name="s",
                                num_cores=NC),
    compiler_params=pltpu.CompilerParams(needs_layout_passes=False,
                                         use_tc_tiling_on_sc=False),
    scratch_types=(
        [pltpu.VMEM((CHUNK,), jnp.float32) for _ in range(9)]
        + [pltpu.VMEM((KSUB, SUB), jnp.int32) for _ in range(6)]
        + [pltpu.VMEM((CHUNK, 8), jnp.float32) for _ in range(3)]
        + [pltpu.VMEM((CHUNK, 16), jnp.float32) for _ in range(3)]
        + [pltpu.SemaphoreType.DMA for _ in range(3)]
        + [pltpu.VMEM_SHARED((NPAD, 8), jnp.float32),
           pltpu.VMEM_SHARED((NPAD, 16), jnp.float32)]
    ),
)(_sc_body)


def _tc_body(accf_ref, accd_ref, pos_ref, batch_ref, strain_ref, vol_ref,
             force_ref, stress_ref):
    accf = accf_ref[0] + accf_ref[1]
    accd = accd_ref[0] + accd_ref[1]
    force_ref[...] = accf[:, :3] + accd[:, :3] + 0.01 * jnp.sin(pos_ref[...])
    b = batch_ref[0, 0]
    oh = (b[None, :] == lax.broadcasted_iota(jnp.int32, (NBATCH, BLK), 0))
    part = jnp.dot(oh.astype(jnp.float32), accd[:, 3:9],
                   preferred_element_type=jnp.float32)
    vol = vol_ref[...]

    @pl.when(pl.program_id(0) == 0)
    def _():
        s9 = strain_ref[...]
        lr = jnp.concatenate(
            [s9[:, 0:1], s9[:, 4:5], s9[:, 8:9],
             s9[:, 1:2], s9[:, 5:6], s9[:, 2:3]], axis=1)
        stress_ref[...] = lr * (-2.0) / vol

    stress_ref[...] += -part / vol


_tc_epilogue = pl.pallas_call(
    _tc_body,
    grid=(NBLK,),
    in_specs=[
        pl.BlockSpec((NC, BLK, 8), lambda i: (0, i, 0)),
        pl.BlockSpec((NC, BLK, 16), lambda i: (0, i, 0)),
        pl.BlockSpec((BLK, 3), lambda i: (i, 0)),
        pl.BlockSpec((1, 1, BLK), lambda i: (i, 0, 0)),
        pl.BlockSpec((NBATCH, 9), lambda i: (0, 0)),
        pl.BlockSpec((NBATCH, 1), lambda i: (0, 0)),
    ],
    out_specs=[
        pl.BlockSpec((BLK, 3), lambda i: (i, 0)),
        pl.BlockSpec((NBATCH, 6), lambda i: (0, 0)),
    ],
    out_shape=[
        jax.ShapeDtypeStruct((N_NODES, 3), jnp.float32),
        jax.ShapeDtypeStruct((NBATCH, 6), jnp.float32),
    ],
)


def kernel(edge_vec, edge_index, pos, les_strain, batch, cell_volume):
    evt = edge_vec.T
    ei = edge_index.astype(jnp.int32)
    z8 = jnp.zeros((RPT, 8), jnp.float32)
    z16 = jnp.zeros((RPT, 16), jnp.float32)
    accf, accd = _sc_scatter(evt, ei, z8, z16)

    pos_pad = jnp.pad(pos, ((0, NPAD - N_NODES), (0, 0)))
    batch_pad = jnp.pad(batch.astype(jnp.int32),
                        (0, NPAD - N_NODES)).reshape(NBLK, 1, BLK)
    strain9 = les_strain.reshape(NBATCH, 9)
    vol2 = cell_volume.reshape(NBATCH, 1)
    force, stress = _tc_epilogue(accf, accd, pos_pad, batch_pad,
                                 strain9, vol2)
    return force, stress
